# baseline (device time: 109152 ns/iter reference)
import jax
import jax.numpy as jnp
from jax import lax
from jax.experimental import pallas as pl
from jax.experimental.pallas import tpu as pltpu

N_DEV = 8
N_BF = 3
AXES = ((1, 2, 4), (2, 4, 1), (4, 1, 2))
BASES = (0, 640, 1280)
SIZES = (640, 640, 768)

_sem_signal = getattr(pl, "semaphore_signal", None) or pltpu.semaphore_signal
_sem_wait = getattr(pl, "semaphore_wait", None) or pltpu.semaphore_wait
_DevId = getattr(pl, "DeviceIdType", None) or pltpu.DeviceIdType
_CompilerParams = getattr(pltpu, "CompilerParams", None) or getattr(
    pltpu, "TPUCompilerParams"
)


def kernel(A, B):
    M, K = A.shape
    N = B.shape[1]
    NC = N // 2

    A16 = A.astype(jnp.bfloat16)
    B16 = B.astype(jnp.bfloat16)

    def body(a_ref, b_ref, out_ref, z_ref,
             st0, st1, st2,
             c00, c10, c20, c01, c11, c21, c02, c12, c22,
             send_sems, recv_sems):
        stages = (st0, st1, st2)
        comms = ((c00, c01, c02), (c10, c11, c12), (c20, c21, c22))

        m = lax.axis_index("i")
        L = m ^ ((m >> 1) & 1)

        def partner(ab):
            pL = L ^ ab
            return pL ^ ((pL >> 1) & 1)

        barrier = pltpu.get_barrier_semaphore()
        for ab in (1, 2, 4):
            _sem_signal(barrier, inc=1, device_id=(partner(ab),),
                        device_id_type=_DevId.MESH)
        _sem_wait(barrier, 3)

        offs = [jnp.int32(BASES[b]) for b in range(N_BF)]
        szs = [SIZES[b] for b in range(N_BF)]

        def rs_halves(b, k):
            ab = AXES[b][k]
            half = szs[b] // 2
            keep_lower = (L & ab) == 0
            send_off = offs[b] + jnp.where(keep_lower, half, 0)
            keep_off = offs[b] + jnp.where(keep_lower, 0, half)
            return ab, half, send_off, keep_off

        def rs_start(b, k, c, half, send_off, ab):
            stages[b][:half, c * NC:(c + 1) * NC] = z_ref[
                pl.ds(send_off, half), c * NC:(c + 1) * NC
            ].astype(jnp.bfloat16)
            rdma = pltpu.make_async_remote_copy(
                src_ref=stages[b].at[:half, c * NC:(c + 1) * NC],
                dst_ref=comms[b][k].at[:, c * NC:(c + 1) * NC],
                send_sem=send_sems.at[b, k, c],
                recv_sem=recv_sems.at[b, k, c],
                device_id=(partner(ab),),
                device_id_type=_DevId.MESH,
            )
            rdma.start()
            return rdma

        def rs_finish(rdma, b, k, c, half, keep_off):
            rdma.wait()
            z_ref[pl.ds(keep_off, half), c * NC:(c + 1) * NC] = (
                z_ref[pl.ds(keep_off, half), c * NC:(c + 1) * NC]
                + comms[b][k][:, c * NC:(c + 1) * NC]
            )

        started = []
        meta = []
        for b in range(N_BF):
            z_ref[pl.ds(BASES[b], SIZES[b]), :] = jnp.dot(
                a_ref[pl.ds(BASES[b], SIZES[b]), :], b_ref[...],
                preferred_element_type=jnp.float32)
            ab, half, send_off, keep_off = rs_halves(b, 0)
            meta.append((half, keep_off))
            for c in range(2):
                started.append((rs_start(b, 0, c, half, send_off, ab), b, c))
        for rdma, b, c in started:
            rs_finish(rdma, b, 0, c, meta[b][0], meta[b][1])
        for b in range(N_BF):
            offs[b] = meta[b][1]
            szs[b] = meta[b][0]

        for k in (1, 2):
            started = []
            meta = []
            for b in range(N_BF):
                ab, half, send_off, keep_off = rs_halves(b, k)
                meta.append((half, keep_off, send_off, ab))
            for c in range(2):
                for b in range(N_BF):
                    half, keep_off, send_off, ab = meta[b]
                    started.append(
                        (rs_start(b, k, c, half, send_off, ab), b, c))
            for rdma, b, c in started:
                rs_finish(rdma, b, k, c, meta[b][0], meta[b][1])
            for b in range(N_BF):
                offs[b] = meta[b][1]
                szs[b] = meta[b][0]

        for b in range(N_BF):
            zc = z_ref[pl.ds(offs[b], szs[b]), :]
            out_ref[pl.ds(offs[b], szs[b]), :] = (
                zc / (1.0 + jnp.exp(-zc))).astype(jnp.bfloat16)

        for j in range(3):
            k = 3 + j
            started = []
            for b in range(N_BF):
                ab = AXES[b][2 - j]
                rdma = pltpu.make_async_remote_copy(
                    src_ref=out_ref.at[pl.ds(offs[b], szs[b]), :],
                    dst_ref=out_ref.at[pl.ds(offs[b], szs[b]), :],
                    send_sem=send_sems.at[b, k, 0],
                    recv_sem=recv_sems.at[b, k, 0],
                    device_id=(partner(ab),),
                    device_id_type=_DevId.MESH,
                )
                rdma.start()
                started.append((rdma, b, ab))
            for rdma, b, ab in started:
                rdma.wait()
                keep_lower = (L & ab) == 0
                offs[b] = offs[b] - jnp.where(keep_lower, 0, szs[b])
                szs[b] = szs[b] * 2

    comm_shapes = [
        pltpu.VMEM((SIZES[b] >> (k + 1), N), jnp.bfloat16)
        for k in range(3) for b in range(N_BF)
    ]
    return pl.pallas_call(
        body,
        out_shape=jax.ShapeDtypeStruct((M, N), jnp.bfloat16),
        in_specs=[
            pl.BlockSpec(memory_space=pltpu.VMEM),
            pl.BlockSpec(memory_space=pltpu.VMEM),
        ],
        out_specs=pl.BlockSpec(memory_space=pltpu.VMEM),
        scratch_shapes=[
            pltpu.VMEM((M, N), jnp.float32),
            pltpu.VMEM((SIZES[0] // 2, N), jnp.bfloat16),
            pltpu.VMEM((SIZES[1] // 2, N), jnp.bfloat16),
            pltpu.VMEM((SIZES[2] // 2, N), jnp.bfloat16),
            *comm_shapes,
            pltpu.SemaphoreType.DMA((N_BF, 6, 2)),
            pltpu.SemaphoreType.DMA((N_BF, 6, 2)),
        ],
        compiler_params=_CompilerParams(
            collective_id=0,
            vmem_limit_bytes=60 * 1024 * 1024,
        ),
    )(A16, B16)


# device time: 99974 ns/iter; 1.0918x vs baseline; 1.0918x over previous
import os

import jax
import jax.numpy as jnp
from jax import lax
from jax.experimental import pallas as pl
from jax.experimental.pallas import tpu as pltpu

_NOCOMM = os.environ.get("ABLATE", "") == "nocomm"

N_DEV = 8
N_BF = 3
AXES = ((1, 2, 4), (2, 4, 1), (4, 1, 2))
BASES = (0, 640, 1280)
SIZES = (640, 640, 768)

_sem_signal = getattr(pl, "semaphore_signal", None) or pltpu.semaphore_signal
_sem_wait = getattr(pl, "semaphore_wait", None) or pltpu.semaphore_wait
_DevId = getattr(pl, "DeviceIdType", None) or pltpu.DeviceIdType
_CompilerParams = getattr(pltpu, "CompilerParams", None) or getattr(
    pltpu, "TPUCompilerParams"
)


def kernel(A, B):
    M, K = A.shape
    N = B.shape[1]
    NC = N // 2

    def body(a_ref, b_ref, out_ref, z_ref, a16, b16,
             c00, c10, c20, c01, c11, c21, c02, c12, c22,
             send_sems, recv_sems):
        comms = ((c00, c01, c02), (c10, c11, c12), (c20, c21, c22))

        m = lax.axis_index("i")
        L = m ^ ((m >> 1) & 1)

        def partner(ab):
            pL = L ^ ab
            return pL ^ ((pL >> 1) & 1)

        barrier = pltpu.get_barrier_semaphore()
        for ab in (1, 2, 4):
            _sem_signal(barrier, inc=1, device_id=(partner(ab),),
                        device_id_type=_DevId.MESH)
        _sem_wait(barrier, 3)

        b16[...] = b_ref[...].astype(jnp.bfloat16)

        offs = [jnp.int32(BASES[b]) for b in range(N_BF)]
        szs = [SIZES[b] for b in range(N_BF)]

        def rs_halves(b, k):
            ab = AXES[b][k]
            half = szs[b] // 2
            keep_lower = (L & ab) == 0
            send_off = offs[b] + jnp.where(keep_lower, half, 0)
            keep_off = offs[b] + jnp.where(keep_lower, 0, half)
            return ab, half, send_off, keep_off

        def rs_start(b, k, c, half, send_off, ab):
            if _NOCOMM:
                return None
            rdma = pltpu.make_async_remote_copy(
                src_ref=z_ref.at[pl.ds(send_off, half), c * NC:(c + 1) * NC],
                dst_ref=comms[b][k].at[:, c * NC:(c + 1) * NC],
                send_sem=send_sems.at[b, k, c],
                recv_sem=recv_sems.at[b, k, c],
                device_id=(partner(ab),),
                device_id_type=_DevId.MESH,
            )
            rdma.start()
            return rdma

        def rs_finish(rdma, b, k, c, half, keep_off):
            if rdma is not None:
                rdma.wait()
            z_ref[pl.ds(keep_off, half), c * NC:(c + 1) * NC] = (
                z_ref[pl.ds(keep_off, half), c * NC:(c + 1) * NC]
                + comms[b][k][:, c * NC:(c + 1) * NC]
            )

        started = []
        meta = []
        for b in range(N_BF):
            rows = pl.ds(BASES[b], SIZES[b])
            a16[rows, :] = a_ref[rows, :].astype(jnp.bfloat16)
            z_ref[rows, :] = jnp.dot(
                a16[rows, :], b16[...], preferred_element_type=jnp.float32
            ).astype(jnp.bfloat16)
            ab, half, send_off, keep_off = rs_halves(b, 0)
            meta.append((half, keep_off))
            for c in range(2):
                started.append((rs_start(b, 0, c, half, send_off, ab), b, c))
        for rdma, b, c in started:
            rs_finish(rdma, b, 0, c, meta[b][0], meta[b][1])
        for b in range(N_BF):
            offs[b] = meta[b][1]
            szs[b] = meta[b][0]

        for k in (1, 2):
            started = []
            meta = []
            for b in range(N_BF):
                ab, half, send_off, keep_off = rs_halves(b, k)
                meta.append((half, keep_off, send_off, ab))
            for c in range(2):
                for b in range(N_BF):
                    half, keep_off, send_off, ab = meta[b]
                    started.append(
                        (rs_start(b, k, c, half, send_off, ab), b, c))
            for rdma, b, c in started:
                rs_finish(rdma, b, k, c, meta[b][0], meta[b][1])
            for b in range(N_BF):
                offs[b] = meta[b][1]
                szs[b] = meta[b][0]

        for b in range(N_BF):
            zc = z_ref[pl.ds(offs[b], szs[b]), :].astype(jnp.float32)
            out_ref[pl.ds(offs[b], szs[b]), :] = (
                zc / (1.0 + jnp.exp(-zc))).astype(jnp.bfloat16)

        for j in range(3):
            k = 3 + j
            started = []
            for b in range(N_BF):
                ab = AXES[b][2 - j]
                if _NOCOMM:
                    started.append((None, b, ab))
                    continue
                rdma = pltpu.make_async_remote_copy(
                    src_ref=out_ref.at[pl.ds(offs[b], szs[b]), :],
                    dst_ref=out_ref.at[pl.ds(offs[b], szs[b]), :],
                    send_sem=send_sems.at[b, k, 0],
                    recv_sem=recv_sems.at[b, k, 0],
                    device_id=(partner(ab),),
                    device_id_type=_DevId.MESH,
                )
                rdma.start()
                started.append((rdma, b, ab))
            for rdma, b, ab in started:
                if rdma is not None:
                    rdma.wait()
                keep_lower = (L & ab) == 0
                offs[b] = offs[b] - jnp.where(keep_lower, 0, szs[b])
                szs[b] = szs[b] * 2

    comm_shapes = [
        pltpu.VMEM((SIZES[b] >> (k + 1), N), jnp.bfloat16)
        for k in range(3) for b in range(N_BF)
    ]
    return pl.pallas_call(
        body,
        out_shape=jax.ShapeDtypeStruct((M, N), jnp.bfloat16),
        in_specs=[
            pl.BlockSpec(memory_space=pltpu.VMEM),
            pl.BlockSpec(memory_space=pltpu.VMEM),
        ],
        out_specs=pl.BlockSpec(memory_space=pltpu.VMEM),
        scratch_shapes=[
            pltpu.VMEM((M, N), jnp.bfloat16),
            pltpu.VMEM((M, K), jnp.bfloat16),
            pltpu.VMEM((K, N), jnp.bfloat16),
            *comm_shapes,
            pltpu.SemaphoreType.DMA((N_BF, 6, 2)),
            pltpu.SemaphoreType.DMA((N_BF, 6, 2)),
        ],
        compiler_params=_CompilerParams(
            collective_id=0,
            vmem_limit_bytes=62 * 1024 * 1024,
        ),
    )(A, B)
